# baseline (speedup 1.0000x reference)
import functools

import jax
import jax.numpy as jnp
from jax import lax
from jax.experimental import pallas as pl
from jax.experimental.pallas import tpu as pltpu
from jax.experimental.pallas import tpu_sc as plsc

N = 10000
E = 320000
D = 128

NC, NS = 2, 16
NW = NC * NS
NPAD = 10240
EPAD = 327680
CH = 128
GP = EPAD // NW // CH
CHP = 64
GPP = EPAD // NW // CHP
QHP = GPP // 4
RPT = NPAD // NS

_MESH = dict(core_axis_name="c", subcore_axis_name="s")


def _deg_body(src_hbm, dst_hbm, out_hbm, degs, degd, srcidx, dstidx,
              ones_v, stage, sem):
    c = lax.axis_index("c")
    s = lax.axis_index("s")
    w = c * NS + s

    zv = jnp.zeros((16,), jnp.float32)
    ov = jnp.ones((16,), jnp.float32)
    for j in range(CH // 16):
        ones_v[pl.ds(j * 16, 16)] = ov
    for j in range(RPT // 16):
        stage[0, pl.ds(j * 16, 16)] = zv
        stage[1, pl.ds(j * 16, 16)] = zv
    pltpu.sync_copy(stage.at[0], degs.at[pl.ds(s * RPT, RPT)])
    pltpu.sync_copy(stage.at[1], degd.at[pl.ds(s * RPT, RPT)])
    plsc.subcore_barrier()

    pltpu.sync_copy(src_hbm.at[pl.ds(w * GP, GP)], srcidx)
    pltpu.sync_copy(dst_hbm.at[pl.ds(w * GP, GP)], dstidx)

    def body(i, carry):
        g = i * 2
        d1 = pltpu.async_copy(ones_v, degs.at[srcidx.at[g]], sem, add=True)
        d2 = pltpu.async_copy(ones_v, degd.at[dstidx.at[g]], sem, add=True)
        d3 = pltpu.async_copy(ones_v, degs.at[srcidx.at[g + 1]], sem, add=True)
        d4 = pltpu.async_copy(ones_v, degd.at[dstidx.at[g + 1]], sem, add=True)
        d1.wait()
        d2.wait()
        d3.wait()
        d4.wait()
        return carry

    lax.fori_loop(0, GP // 2, body, 0)
    plsc.subcore_barrier()

    pltpu.sync_copy(degs.at[pl.ds(s * RPT, RPT)], stage.at[0])
    pltpu.sync_copy(degd.at[pl.ds(s * RPT, RPT)], stage.at[1])
    pltpu.sync_copy(stage, out_hbm.at[w])


@functools.partial(jax.jit, static_argnames=())
def _deg_call(srcp, dstp):
    k = pl.kernel(
        _deg_body,
        out_type=jax.ShapeDtypeStruct((NW, 2, RPT), jnp.float32),
        mesh=plsc.VectorSubcoreMesh(**_MESH),
        scratch_types=[
            pltpu.VMEM_SHARED((NPAD,), jnp.float32),
            pltpu.VMEM_SHARED((NPAD,), jnp.float32),
            pltpu.VMEM((GP, CH), jnp.int32),
            pltpu.VMEM((GP, CH), jnp.int32),
            pltpu.VMEM((CH,), jnp.float32),
            pltpu.VMEM((2, RPT), jnp.float32),
            pltpu.SemaphoreType.DMA,
        ],
    )
    return k(srcp, dstp)


def _prop_body(z_hbm, src_hbm, dst_hbm, out_hbm, accum, srcidx, dstidx,
               rows0, rows1, rows2, rows3, gs0, gs1, gs2, gs3,
               ss0, ss1, ss2, ss3):
    c = lax.axis_index("c")
    s = lax.axis_index("s")
    w = c * NS + s
    rows = (rows0, rows1, rows2, rows3)
    gsem = (gs0, gs1, gs2, gs3)
    ssem = (ss0, ss1, ss2, ss3)

    zv = jnp.zeros((16,), jnp.float32)

    def zrow(i, carry):
        for j in range(D // 16):
            rows0[i, pl.ds(j * 16, 16)] = zv
        return carry

    lax.fori_loop(0, CHP, zrow, 0)
    for j in range(RPT // CHP):
        pltpu.sync_copy(rows0, accum.at[pl.ds(s * RPT + j * CHP, CHP)])
    plsc.subcore_barrier()

    def gstart(g, b):
        pltpu.async_copy(z_hbm.at[srcidx.at[g]], rows[b], gsem[b])

    def gwait(b):
        pltpu.make_async_copy(z_hbm.at[srcidx.at[0]], rows[b], gsem[b]).wait()

    def sstart(g, b):
        pltpu.async_copy(rows[b], accum.at[dstidx.at[g]], ssem[b], add=True)

    def swait(b):
        pltpu.make_async_copy(rows[b], accum.at[dstidx.at[0]], ssem[b]).wait()

    for q in range(GPP // QHP):
        pltpu.sync_copy(src_hbm.at[pl.ds(w * GPP + q * QHP, QHP)], srcidx)
        pltpu.sync_copy(dst_hbm.at[pl.ds(w * GPP + q * QHP, QHP)], dstidx)
        for b in range(3):
            gstart(b, b)

        def group(i4, carry):
            for b in range(4):
                g = i4 * 4 + b
                gwait(b)
                sstart(g, b)
                b3 = (b + 3) % 4

                @pl.when(g + 3 < QHP)
                def _():
                    @pl.when(g >= 1)
                    def _():
                        swait(b3)

                    gstart(g + 3, b3)

            return carry

        lax.fori_loop(0, QHP // 4, group, 0)
        for b in range(4):
            swait(b)
    plsc.subcore_barrier()

    pltpu.sync_copy(accum.at[pl.ds(s * RPT, RPT)], out_hbm.at[w])


@jax.jit
def _prop_call(z, srcp, dstp):
    k = pl.kernel(
        _prop_body,
        out_type=jax.ShapeDtypeStruct((NW, RPT, D), jnp.float32),
        mesh=plsc.VectorSubcoreMesh(**_MESH),
        scratch_types=[
            pltpu.VMEM_SHARED((NPAD, D), jnp.float32),
            pltpu.VMEM((QHP, CHP), jnp.int32),
            pltpu.VMEM((QHP, CHP), jnp.int32),
            pltpu.VMEM((CHP, D), jnp.float32),
            pltpu.VMEM((CHP, D), jnp.float32),
            pltpu.VMEM((CHP, D), jnp.float32),
            pltpu.VMEM((CHP, D), jnp.float32),
            pltpu.SemaphoreType.DMA,
            pltpu.SemaphoreType.DMA,
            pltpu.SemaphoreType.DMA,
            pltpu.SemaphoreType.DMA,
            pltpu.SemaphoreType.DMA,
            pltpu.SemaphoreType.DMA,
            pltpu.SemaphoreType.DMA,
            pltpu.SemaphoreType.DMA,
        ],
    )
    return k(z, srcp, dstp)


R = 512
NBLK = NPAD // R


def _mm1_body(h_ref, w_ref, deg_ref, z_ref):
    d = deg_ref[...]
    ns = lax.rsqrt(jnp.maximum(d[0, 0] + d[1, 0], 1.0))
    acc = lax.dot_general(h_ref[...], w_ref[...], (((0,), (0,)), ((), ())),
                          preferred_element_type=jnp.float32)
    z_ref[...] = acc * ns[:, None]


def _mid_body(p_ref, deg_ref, w_ref, b_ref, z_ref):
    d = deg_ref[...]
    ns = lax.rsqrt(jnp.maximum(d[0, 0] + d[1, 0], 1.0))
    nd = lax.rsqrt(jnp.maximum(d[0, 1] + d[1, 1], 1.0))
    p = p_ref[0] + p_ref[1]
    u = jnp.maximum(p * nd[:, None] + b_ref[...], 0.0)
    z_ref[...] = jnp.dot(u, w_ref[...], preferred_element_type=jnp.float32) * ns[:, None]


def _post_body(p_ref, deg_ref, b_ref, o_ref):
    d = deg_ref[...]
    nd = lax.rsqrt(jnp.maximum(d[0, 1] + d[1, 1], 1.0))
    o_ref[...] = jnp.transpose((p_ref[0] + p_ref[1]) * nd[:, None] + b_ref[...])


_DEG_SPEC = pl.BlockSpec((2, 2, R), lambda i: (0, 0, i))


@jax.jit
def _mm1_call(hp, W1, degp):
    return pl.pallas_call(
        _mm1_body,
        grid=(NBLK,),
        in_specs=[
            pl.BlockSpec((D, R), lambda i: (0, i)),
            pl.BlockSpec((D, D), lambda i: (0, 0)),
            _DEG_SPEC,
        ],
        out_specs=pl.BlockSpec((R, D), lambda i: (i, 0)),
        out_shape=jax.ShapeDtypeStruct((NPAD, D), jnp.float32),
    )(hp, W1, degp)


@jax.jit
def _mid_call(p1, degp, W2, b1):
    return pl.pallas_call(
        _mid_body,
        grid=(NBLK,),
        in_specs=[
            pl.BlockSpec((2, R, D), lambda i: (0, i, 0)),
            _DEG_SPEC,
            pl.BlockSpec((D, D), lambda i: (0, 0)),
            pl.BlockSpec((1, D), lambda i: (0, 0)),
        ],
        out_specs=pl.BlockSpec((R, D), lambda i: (i, 0)),
        out_shape=jax.ShapeDtypeStruct((NPAD, D), jnp.float32),
    )(p1, degp, W2, b1)


@jax.jit
def _post_call(p2, degp, b2):
    return pl.pallas_call(
        _post_body,
        grid=(NBLK,),
        in_specs=[
            pl.BlockSpec((2, R, D), lambda i: (0, i, 0)),
            _DEG_SPEC,
            pl.BlockSpec((1, D), lambda i: (0, 0)),
        ],
        out_specs=pl.BlockSpec((D, R), lambda i: (0, i)),
        out_shape=jax.ShapeDtypeStruct((D, NPAD), jnp.float32),
    )(p2, degp, b2)


def kernel(h, edge_index, W1, b1, W2, b2):
    hp = jnp.pad(h, ((0, 0), (0, NPAD - N)))
    src = edge_index[0]
    dst = edge_index[1]
    fill = (jnp.arange(EPAD - E, dtype=jnp.int32) % (NPAD - N)) + N
    srcf = jnp.concatenate([src, fill])
    dstf = jnp.concatenate([dst, fill])
    srcp = srcf.reshape(EPAD // CH, CH)
    dstp = dstf.reshape(EPAD // CH, CH)
    srcq = srcf.reshape(EPAD // CHP, CHP)
    dstq = dstf.reshape(EPAD // CHP, CHP)

    dk = _deg_call(srcp, dstp)
    degp = dk.reshape(NC, NS, 2, RPT).transpose(0, 2, 1, 3).reshape(NC, 2, NPAD)

    z1 = _mm1_call(hp, W1, degp)
    p1 = _prop_call(z1, srcq, dstq).reshape(NC, NPAD, D)
    z2 = _mid_call(p1, degp, W2, b1.reshape(1, D))
    p2 = _prop_call(z2, srcq, dstq).reshape(NC, NPAD, D)
    o = _post_call(p2, degp, b2.reshape(1, D))
    return o[:, :N]


# speedup vs baseline: 1.0294x; 1.0294x over previous
import functools

import jax
import jax.numpy as jnp
from jax import lax
from jax.experimental import pallas as pl
from jax.experimental.pallas import tpu as pltpu
from jax.experimental.pallas import tpu_sc as plsc

N = 10000
E = 320000
D = 128

NC, NS = 2, 16
NW = NC * NS
CH = 128
NPAD = 10240
EPAD = 327680
GP = EPAD // NW // CH
QH = GP // 2
RPT = NPAD // NS

_MESH = dict(core_axis_name="c", subcore_axis_name="s")


def _deg_body(src_hbm, dst_hbm, out_hbm, degs, degd, srcidx, dstidx,
              ones_v, stage, sem):
    c = lax.axis_index("c")
    s = lax.axis_index("s")
    w = c * NS + s

    zv = jnp.zeros((16,), jnp.float32)
    ov = jnp.ones((16,), jnp.float32)
    for j in range(CH // 16):
        ones_v[pl.ds(j * 16, 16)] = ov
    for j in range(RPT // 16):
        stage[0, pl.ds(j * 16, 16)] = zv
        stage[1, pl.ds(j * 16, 16)] = zv
    pltpu.sync_copy(stage.at[0], degs.at[pl.ds(s * RPT, RPT)])
    pltpu.sync_copy(stage.at[1], degd.at[pl.ds(s * RPT, RPT)])
    plsc.subcore_barrier()

    pltpu.sync_copy(src_hbm.at[pl.ds(w * GP, GP)], srcidx)
    pltpu.sync_copy(dst_hbm.at[pl.ds(w * GP, GP)], dstidx)

    def body(i, carry):
        g = i * 2
        d1 = pltpu.async_copy(ones_v, degs.at[srcidx.at[g]], sem, add=True)
        d2 = pltpu.async_copy(ones_v, degd.at[dstidx.at[g]], sem, add=True)
        d3 = pltpu.async_copy(ones_v, degs.at[srcidx.at[g + 1]], sem, add=True)
        d4 = pltpu.async_copy(ones_v, degd.at[dstidx.at[g + 1]], sem, add=True)
        d1.wait()
        d2.wait()
        d3.wait()
        d4.wait()
        return carry

    lax.fori_loop(0, GP // 2, body, 0)
    plsc.subcore_barrier()

    pltpu.sync_copy(degs.at[pl.ds(s * RPT, RPT)], stage.at[0])
    pltpu.sync_copy(degd.at[pl.ds(s * RPT, RPT)], stage.at[1])
    pltpu.sync_copy(stage, out_hbm.at[w])


@functools.partial(jax.jit, static_argnames=())
def _deg_call(srcp, dstp):
    k = pl.kernel(
        _deg_body,
        out_type=jax.ShapeDtypeStruct((NW, 2, RPT), jnp.float32),
        mesh=plsc.VectorSubcoreMesh(**_MESH),
        scratch_types=[
            pltpu.VMEM_SHARED((NPAD,), jnp.float32),
            pltpu.VMEM_SHARED((NPAD,), jnp.float32),
            pltpu.VMEM((GP, CH), jnp.int32),
            pltpu.VMEM((GP, CH), jnp.int32),
            pltpu.VMEM((CH,), jnp.float32),
            pltpu.VMEM((2, RPT), jnp.float32),
            pltpu.SemaphoreType.DMA,
        ],
    )
    return k(srcp, dstp)


def _prop_body(z_hbm, src_hbm, dst_hbm, out_hbm, accum, srcidx, dstidx,
               rows_a, rows_b, sem_a, sem_b):
    c = lax.axis_index("c")
    s = lax.axis_index("s")
    w = c * NS + s

    zv = jnp.zeros((16,), jnp.float32)

    def zrow(i, carry):
        for j in range(D // 16):
            rows_a[i, pl.ds(j * 16, 16)] = zv
        return carry

    lax.fori_loop(0, CH, zrow, 0)
    for j in range(RPT // CH):
        pltpu.sync_copy(rows_a, accum.at[pl.ds(s * RPT + j * CH, CH)])
    plsc.subcore_barrier()

    def gstart(g, buf, sem):
        pltpu.async_copy(z_hbm.at[srcidx.at[g]], buf, sem)

    def gwait(g, buf, sem):
        pltpu.make_async_copy(z_hbm.at[srcidx.at[g]], buf, sem).wait()

    for q in range(GP // QH):
        pltpu.sync_copy(src_hbm.at[pl.ds(w * GP + q * QH, QH)], srcidx)
        pltpu.sync_copy(dst_hbm.at[pl.ds(w * GP + q * QH, QH)], dstidx)
        gstart(0, rows_a, sem_a)

        def body(i, carry):
            g = i * 2
            gstart(g + 1, rows_b, sem_b)
            gwait(g, rows_a, sem_a)
            pltpu.sync_copy(rows_a, accum.at[dstidx.at[g]], add=True)

            @pl.when(g + 2 < QH)
            def _():
                gstart(g + 2, rows_a, sem_a)

            gwait(g + 1, rows_b, sem_b)
            pltpu.sync_copy(rows_b, accum.at[dstidx.at[g + 1]], add=True)
            return carry

        lax.fori_loop(0, QH // 2, body, 0)
    plsc.subcore_barrier()

    pltpu.sync_copy(accum.at[pl.ds(s * RPT, RPT)], out_hbm.at[w])


@jax.jit
def _prop_call(z, srcp, dstp):
    k = pl.kernel(
        _prop_body,
        out_type=jax.ShapeDtypeStruct((NW, RPT, D), jnp.float32),
        mesh=plsc.VectorSubcoreMesh(**_MESH),
        scratch_types=[
            pltpu.VMEM_SHARED((NPAD, D), jnp.float32),
            pltpu.VMEM((QH, CH), jnp.int32),
            pltpu.VMEM((QH, CH), jnp.int32),
            pltpu.VMEM((CH, D), jnp.float32),
            pltpu.VMEM((CH, D), jnp.float32),
            pltpu.SemaphoreType.DMA,
            pltpu.SemaphoreType.DMA,
        ],
    )
    return k(z, srcp, dstp)


R = 512
NBLK = NPAD // R


def _mm1_body(x_ref, w_ref, deg_ref, z_ref):
    d = deg_ref[...]
    ns = lax.rsqrt(jnp.maximum(d[0, 0] + d[1, 0], 1.0))
    acc = jnp.dot(x_ref[...], w_ref[...], preferred_element_type=jnp.float32)
    z_ref[...] = acc * ns[:, None]


def _mid_body(p_ref, deg_ref, w_ref, b_ref, z_ref):
    d = deg_ref[...]
    ns = lax.rsqrt(jnp.maximum(d[0, 0] + d[1, 0], 1.0))
    nd = lax.rsqrt(jnp.maximum(d[0, 1] + d[1, 1], 1.0))
    p = p_ref[0] + p_ref[1]
    u = jnp.maximum(p * nd[:, None] + b_ref[...], 0.0)
    z_ref[...] = jnp.dot(u, w_ref[...], preferred_element_type=jnp.float32) * ns[:, None]


def _post_body(p_ref, deg_ref, b_ref, o_ref):
    d = deg_ref[...]
    nd = lax.rsqrt(jnp.maximum(d[0, 1] + d[1, 1], 1.0))
    o_ref[...] = (p_ref[0] + p_ref[1]) * nd[:, None] + b_ref[...]


_DEG_SPEC = pl.BlockSpec((2, 2, R), lambda i: (0, 0, i))


@jax.jit
def _mm1_call(x, W1, degp):
    return pl.pallas_call(
        _mm1_body,
        grid=(NBLK,),
        in_specs=[
            pl.BlockSpec((R, D), lambda i: (i, 0)),
            pl.BlockSpec((D, D), lambda i: (0, 0)),
            _DEG_SPEC,
        ],
        out_specs=pl.BlockSpec((R, D), lambda i: (i, 0)),
        out_shape=jax.ShapeDtypeStruct((NPAD, D), jnp.float32),
    )(x, W1, degp)


@jax.jit
def _mid_call(p1, degp, W2, b1):
    return pl.pallas_call(
        _mid_body,
        grid=(NBLK,),
        in_specs=[
            pl.BlockSpec((2, R, D), lambda i: (0, i, 0)),
            _DEG_SPEC,
            pl.BlockSpec((D, D), lambda i: (0, 0)),
            pl.BlockSpec((1, D), lambda i: (0, 0)),
        ],
        out_specs=pl.BlockSpec((R, D), lambda i: (i, 0)),
        out_shape=jax.ShapeDtypeStruct((NPAD, D), jnp.float32),
    )(p1, degp, W2, b1)


@jax.jit
def _post_call(p2, degp, b2):
    return pl.pallas_call(
        _post_body,
        grid=(NBLK,),
        in_specs=[
            pl.BlockSpec((2, R, D), lambda i: (0, i, 0)),
            _DEG_SPEC,
            pl.BlockSpec((1, D), lambda i: (0, 0)),
        ],
        out_specs=pl.BlockSpec((R, D), lambda i: (i, 0)),
        out_shape=jax.ShapeDtypeStruct((NPAD, D), jnp.float32),
    )(p2, degp, b2)


def kernel(h, edge_index, W1, b1, W2, b2):
    x = jnp.transpose(h)
    x = jnp.pad(x, ((0, NPAD - N), (0, 0)))
    src = edge_index[0]
    dst = edge_index[1]
    fill = (jnp.arange(EPAD - E, dtype=jnp.int32) % (NPAD - N)) + N
    srcp = jnp.concatenate([src, fill]).reshape(EPAD // CH, CH)
    dstp = jnp.concatenate([dst, fill]).reshape(EPAD // CH, CH)

    dk = _deg_call(srcp, dstp)
    degp = dk.reshape(NC, NS, 2, RPT).transpose(0, 2, 1, 3).reshape(NC, 2, NPAD)

    z1 = _mm1_call(x, W1, degp)
    p1 = _prop_call(z1, srcp, dstp).reshape(NC, NPAD, D)
    z2 = _mid_call(p1, degp, W2, b1.reshape(1, D))
    p2 = _prop_call(z2, srcp, dstp).reshape(NC, NPAD, D)
    o = _post_call(p2, degp, b2.reshape(1, D))
    return jnp.transpose(o[:N])
